# baseline (speedup 1.0000x reference)
import functools

import jax
import jax.numpy as jnp
from jax import lax
from jax.experimental import pallas as pl
from jax.experimental.pallas import tpu as pltpu
from jax.experimental.pallas import tpu_sc as plsc

N = 10000
E = 320000
D = 128

NC = 2
NS = 16
NW = NC * NS

CHUNK = 50
E_PER_TILE = E // NW
CHUNKS_PER_TILE = E_PER_TILE // CHUNK
IBLOCKS = 8
CPB = CHUNKS_PER_TILE // IBLOCKS
NBUF = 5
ZCHUNK = 80
N_ZCHUNKS = N // ZCHUNK
ZITERS = (N_ZCHUNKS + NS - 1) // NS


def _sc_agg_body(h_hbm, src_hbm, dst_hbm, zeros_hbm, out_hbm,
                 src_all, dst_all, rows0, rows1, rows2, rows3, rows4, agg_sh,
                 gsem0, gsem1, gsem2, gsem3, gsem4,
                 ssem0, ssem1, ssem2, ssem3, ssem4):
    cid = lax.axis_index("c")
    sid = lax.axis_index("s")
    wid = cid * NS + sid

    rows = (rows0, rows1, rows2, rows3, rows4)
    gsem = (gsem0, gsem1, gsem2, gsem3, gsem4)
    ssem = (ssem0, ssem1, ssem2, ssem3, ssem4)

    def load_block(b):
        pltpu.sync_copy(src_hbm.at[wid, b], src_all)
        pltpu.sync_copy(dst_hbm.at[wid, b], dst_all)

    def start_gather(c, i):
        pltpu.async_copy(h_hbm.at[src_all.at[c]], rows[i], gsem[i])

    def wait_gather(c, i):
        pltpu.make_async_copy(h_hbm.at[src_all.at[c]], rows[i], gsem[i]).wait()

    def start_scatter(c, i):
        pltpu.async_copy(rows[i], agg_sh.at[dst_all.at[c]], ssem[i], add=True)

    def wait_scatter(c, i):
        pltpu.make_async_copy(rows[i], agg_sh.at[dst_all.at[c]],
                              ssem[i]).wait()

    load_block(0)
    start_gather(0, 0)
    start_gather(1, 1)

    def zbody(k, carry):
        chunk = k * NS + sid

        @pl.when(chunk < N_ZCHUNKS)
        def _():
            pltpu.sync_copy(zeros_hbm, agg_sh.at[pl.ds(chunk * ZCHUNK, ZCHUNK)])

        return carry

    lax.fori_loop(0, ZITERS, zbody, None)
    plsc.subcore_barrier()

    def block_body(b, carry):
        def quint(k, carry2):
            for j in range(NBUF):
                c = NBUF * k + j
                i = j
                wait_gather(c, i)
                start_scatter(c, i)

                nxt = c + 2
                ni = (j + 2) % NBUF
                if j < 3:
                    @pl.when(c >= 3)
                    def _(c=c, ni=ni):
                        wait_scatter(c - 3, ni)

                    start_gather(nxt, ni)
                else:
                    @pl.when(nxt < CPB)
                    def _(nxt=nxt, ni=ni, c=c):
                        wait_scatter(c - 3, ni)
                        start_gather(nxt, ni)
            return carry2

        lax.fori_loop(0, CPB // NBUF, quint, None)

        for t in range(CPB - NBUF, CPB):
            wait_scatter(t, t % NBUF)

        @pl.when(b + 1 < IBLOCKS)
        def _():
            load_block(b + 1)
            start_gather(0, 0)
            start_gather(1, 1)

        return carry

    lax.fori_loop(0, IBLOCKS, block_body, None)
    plsc.subcore_barrier()

    def dbody(k, carry):
        chunk = k * NS + sid

        @pl.when(chunk < N_ZCHUNKS)
        def _():
            pltpu.sync_copy(agg_sh.at[pl.ds(chunk * ZCHUNK, ZCHUNK)],
                            out_hbm.at[cid, pl.ds(chunk * ZCHUNK, ZCHUNK)])

        return carry

    lax.fori_loop(0, ZITERS, dbody, None)


@functools.partial(jax.jit, static_argnames=())
def _sc_agg(h, src2d, dst2d, zeros_tile):
    mesh = plsc.VectorSubcoreMesh(core_axis_name="c", subcore_axis_name="s")
    fn = pl.kernel(
        _sc_agg_body,
        out_type=jax.ShapeDtypeStruct((NC, N, D), jnp.float32),
        mesh=mesh,
        scratch_types=[
            pltpu.VMEM((CPB, CHUNK), jnp.int32),
            pltpu.VMEM((CPB, CHUNK), jnp.int32),
            pltpu.VMEM((CHUNK, D), jnp.float32),
            pltpu.VMEM((CHUNK, D), jnp.float32),
            pltpu.VMEM((CHUNK, D), jnp.float32),
            pltpu.VMEM((CHUNK, D), jnp.float32),
            pltpu.VMEM((CHUNK, D), jnp.float32),
            pltpu.VMEM_SHARED((N, D), jnp.float32),
        ] + [pltpu.SemaphoreType.DMA] * 10,
    )
    return fn(h, src2d, dst2d, zeros_tile)


def _dense_body(relu, p0_ref, p1_ref, x_ref, wr_ref, b_ref, wroot_ref, o_ref):
    agg = p0_ref[...] + p1_ref[...]
    acc = lax.dot_general(agg, wr_ref[...], (((1,), (1,)), ((), ())),
                          preferred_element_type=jnp.float32)
    acc = acc + lax.dot_general(x_ref[...], wroot_ref[...],
                                (((1,), (1,)), ((), ())),
                                preferred_element_type=jnp.float32)
    acc = acc + b_ref[...]
    if relu:
        acc = jnp.maximum(acc, 0.0)
    o_ref[...] = acc


def _dense_call(partials, x, w_rel, b_rel, w_root, relu):
    blk = 1000
    grid = N // blk
    body = functools.partial(_dense_body, relu)
    p0 = partials[0]
    p1 = partials[1]
    return pl.pallas_call(
        body,
        grid=(grid,),
        in_specs=[
            pl.BlockSpec((blk, D), lambda i: (i, 0)),
            pl.BlockSpec((blk, D), lambda i: (i, 0)),
            pl.BlockSpec((blk, D), lambda i: (i, 0)),
            pl.BlockSpec((D, D), lambda i: (0, 0)),
            pl.BlockSpec((1, D), lambda i: (0, 0)),
            pl.BlockSpec((D, D), lambda i: (0, 0)),
        ],
        out_specs=pl.BlockSpec((blk, D), lambda i: (i, 0)),
        out_shape=jax.ShapeDtypeStruct((N, D), jnp.float32),
    )(p0, p1, x, w_rel, b_rel.reshape(1, D), w_root)


def kernel(x, edge_index, W1_rel, b1_rel, W1_root, W2_rel, b2_rel, W2_root,
           W3_rel, b3_rel, W3_root):
    src2d = edge_index[0].astype(jnp.int32).reshape(NW, IBLOCKS, CPB, CHUNK)
    dst2d = edge_index[1].astype(jnp.int32).reshape(NW, IBLOCKS, CPB, CHUNK)
    zeros_tile = jnp.zeros((ZCHUNK, D), jnp.float32)

    h = x
    for W_rel, b_rel, W_root, relu in (
        (W1_rel, b1_rel, W1_root, True),
        (W2_rel, b2_rel, W2_root, True),
        (W3_rel, b3_rel, W3_root, False),
    ):
        partials = _sc_agg(h, src2d, dst2d, zeros_tile)
        h = _dense_call(partials, h, W_rel, b_rel, W_root, relu)
    return h


# speedup vs baseline: 1.3102x; 1.3102x over previous
import functools

import jax
import jax.numpy as jnp
from jax import lax
from jax.experimental import pallas as pl
from jax.experimental.pallas import tpu as pltpu
from jax.experimental.pallas import tpu_sc as plsc

N = 10000
E = 320000
D = 128

NC = 2
NS = 16
NW = NC * NS

CHUNK = 80
E_PER_TILE = E // NW
CHUNKS_PER_TILE = E_PER_TILE // CHUNK
IBLOCKS = 5
CPB = CHUNKS_PER_TILE // IBLOCKS
ZCHUNK = 80
N_ZCHUNKS = N // ZCHUNK
ZITERS = (N_ZCHUNKS + NS - 1) // NS


def _sc_agg_body(h_hbm, src_hbm, dst_hbm, zeros_hbm, out_hbm,
                 src_all, dst_all, rows0, rows1, rows2, agg_sh,
                 gsem0, gsem1, gsem2, ssem0, ssem1, ssem2):
    cid = lax.axis_index("c")
    sid = lax.axis_index("s")
    wid = cid * NS + sid

    rows = (rows0, rows1, rows2)
    gsem = (gsem0, gsem1, gsem2)
    ssem = (ssem0, ssem1, ssem2)

    def load_block(b):
        pltpu.sync_copy(src_hbm.at[wid, b], src_all)
        pltpu.sync_copy(dst_hbm.at[wid, b], dst_all)

    def start_gather(c, i):
        pltpu.async_copy(h_hbm.at[src_all.at[c]], rows[i], gsem[i])

    def wait_gather(c, i):
        pltpu.make_async_copy(h_hbm.at[src_all.at[c]], rows[i], gsem[i]).wait()

    def start_scatter(c, i):
        pltpu.async_copy(rows[i], agg_sh.at[dst_all.at[c]], ssem[i], add=True)

    def wait_scatter(c, i):
        pltpu.make_async_copy(rows[i], agg_sh.at[dst_all.at[c]],
                              ssem[i]).wait()

    load_block(0)
    start_gather(0, 0)
    start_gather(1, 1)

    def zbody(k, carry):
        chunk = k * NS + sid

        @pl.when(chunk < N_ZCHUNKS)
        def _():
            pltpu.sync_copy(zeros_hbm.at[pl.ds(chunk * ZCHUNK, ZCHUNK)],
                            agg_sh.at[pl.ds(chunk * ZCHUNK, ZCHUNK)])

        return carry

    lax.fori_loop(0, ZITERS, zbody, None)
    plsc.subcore_barrier()

    def block_body(b, carry):
        def triple(k, carry2):
            for j in range(3):
                c = 3 * k + j
                i = j

                wait_gather(c, i)
                start_scatter(c, i)

                nxt = c + 2
                ni = (j + 2) % 3
                if j == 2:
                    @pl.when(nxt < CPB)
                    def _(nxt=nxt, ni=ni, c=c):
                        wait_scatter(c - 1, ni)
                        start_gather(nxt, ni)
                elif j == 0:
                    @pl.when(c > 0)
                    def _(c=c, ni=ni):
                        wait_scatter(c - 1, ni)

                    start_gather(nxt, ni)
                else:
                    wait_scatter(c - 1, ni)
                    start_gather(nxt, ni)
            return carry2

        lax.fori_loop(0, CPB // 3, triple, None)

        tail = CPB - 1
        wait_gather(tail, tail % 3)
        start_scatter(tail, tail % 3)
        wait_scatter(tail - 2, (tail - 2) % 3)
        wait_scatter(tail - 1, (tail - 1) % 3)
        wait_scatter(tail, tail % 3)

        @pl.when(b + 1 < IBLOCKS)
        def _():
            load_block(b + 1)
            start_gather(0, 0)
            start_gather(1, 1)

        return carry

    lax.fori_loop(0, IBLOCKS, block_body, None)
    plsc.subcore_barrier()

    def dbody(k, carry):
        chunk = k * NS + sid

        @pl.when(chunk < N_ZCHUNKS)
        def _():
            pltpu.sync_copy(agg_sh.at[pl.ds(chunk * ZCHUNK, ZCHUNK)],
                            out_hbm.at[cid, pl.ds(chunk * ZCHUNK, ZCHUNK)])

        return carry

    lax.fori_loop(0, ZITERS, dbody, None)


@functools.partial(jax.jit, static_argnames=())
def _sc_agg(h, src2d, dst2d, zeros_tile):
    mesh = plsc.VectorSubcoreMesh(core_axis_name="c", subcore_axis_name="s")
    fn = pl.kernel(
        _sc_agg_body,
        out_type=jax.ShapeDtypeStruct((NC, N, D), jnp.float32),
        mesh=mesh,
        scratch_types=[
            pltpu.VMEM((CPB, CHUNK), jnp.int32),
            pltpu.VMEM((CPB, CHUNK), jnp.int32),
            pltpu.VMEM((CHUNK, D), jnp.float32),
            pltpu.VMEM((CHUNK, D), jnp.float32),
            pltpu.VMEM((CHUNK, D), jnp.float32),
            pltpu.VMEM_SHARED((N, D), jnp.float32),
            pltpu.SemaphoreType.DMA,
            pltpu.SemaphoreType.DMA,
            pltpu.SemaphoreType.DMA,
            pltpu.SemaphoreType.DMA,
            pltpu.SemaphoreType.DMA,
            pltpu.SemaphoreType.DMA,
        ],
    )
    return fn(h, src2d, dst2d, zeros_tile)


def _dense_body(relu, p0_ref, p1_ref, x_ref, wr_ref, b_ref, wroot_ref, o_ref):
    agg = p0_ref[...] + p1_ref[...]
    acc = lax.dot_general(agg, wr_ref[...], (((1,), (1,)), ((), ())),
                          preferred_element_type=jnp.float32)
    acc = acc + lax.dot_general(x_ref[...], wroot_ref[...],
                                (((1,), (1,)), ((), ())),
                                preferred_element_type=jnp.float32)
    acc = acc + b_ref[...]
    if relu:
        acc = jnp.maximum(acc, 0.0)
    o_ref[...] = acc


def _dense_call(partials, x, w_rel, b_rel, w_root, relu):
    blk = 1000
    grid = N // blk
    body = functools.partial(_dense_body, relu)
    p0 = partials[0]
    p1 = partials[1]
    return pl.pallas_call(
        body,
        grid=(grid,),
        in_specs=[
            pl.BlockSpec((blk, D), lambda i: (i, 0)),
            pl.BlockSpec((blk, D), lambda i: (i, 0)),
            pl.BlockSpec((blk, D), lambda i: (i, 0)),
            pl.BlockSpec((D, D), lambda i: (0, 0)),
            pl.BlockSpec((1, D), lambda i: (0, 0)),
            pl.BlockSpec((D, D), lambda i: (0, 0)),
        ],
        out_specs=pl.BlockSpec((blk, D), lambda i: (i, 0)),
        out_shape=jax.ShapeDtypeStruct((N, D), jnp.float32),
    )(p0, p1, x, w_rel, b_rel.reshape(1, D), w_root)


def kernel(x, edge_index, W1_rel, b1_rel, W1_root, W2_rel, b2_rel, W2_root,
           W3_rel, b3_rel, W3_root):
    src2d = edge_index[0].astype(jnp.int32).reshape(NW, IBLOCKS, CPB, CHUNK)
    dst2d = edge_index[1].astype(jnp.int32).reshape(NW, IBLOCKS, CPB, CHUNK)
    zeros_tile = jnp.zeros((N, D), jnp.float32)

    h = x
    for W_rel, b_rel, W_root, relu in (
        (W1_rel, b1_rel, W1_root, True),
        (W2_rel, b2_rel, W2_root, True),
        (W3_rel, b3_rel, W3_root, False),
    ):
        partials = _sc_agg(h, src2d, dst2d, zeros_tile)
        h = _dense_call(partials, h, W_rel, b_rel, W_root, relu)
    return h
